# baseline (speedup 1.0000x reference)
import functools

import jax
import jax.numpy as jnp
from jax import lax
from jax.experimental import pallas as pl
from jax.experimental.pallas import tpu as pltpu
from jax.experimental.pallas import tpu_sc as plsc

N_ELEMS = 4096 * 4096
RANK = 15099493
NW = 32
SHARD = N_ELEMS // NW

TOP_BINS = 32768
LOW_BINS = 65536

_SC_PARAMS = pltpu.CompilerParams(needs_layout_passes=False)


def _sc_mesh():
    return plsc.VectorSubcoreMesh(core_axis_name="c", subcore_axis_name="s")


ROWS_PER_TILE = SHARD // 4096
RING = 8
RING2 = 8


def _hist_top(w2d):
    @functools.partial(
        pl.kernel,
        mesh=_sc_mesh(),
        out_type=jax.ShapeDtypeStruct((NW, TOP_BINS), jnp.int32),
        scratch_types=(
            [pltpu.VMEM((4096,), jnp.float32)] * RING
            + [pltpu.VMEM((TOP_BINS,), jnp.int32)]
            + [pltpu.SemaphoreType.DMA] * RING
        ),
        compiler_params=_SC_PARAMS,
    )
    def k(w_hbm, out_hbm, *scratch):
        bufs = scratch[:RING]
        hist = scratch[RING]
        sems = scratch[RING + 1:]
        wid = lax.axis_index("s") * 2 + lax.axis_index("c")
        row0 = wid * ROWS_PER_TILE

        @plsc.parallel_loop(0, TOP_BINS // 16, unroll=8)
        def _zero(i):
            hist[pl.ds(i * 16, 16)] = jnp.zeros((16,), jnp.int32)

        ones = jnp.ones((16,), jnp.int32)
        mask7f = jnp.full((16,), 0x7FFFFFFF, jnp.int32)

        def process(buf):
            @plsc.parallel_loop(0, 4096 // 16, unroll=16)
            def _body(j):
                bits = plsc.bitcast(buf[pl.ds(j * 16, 16)], jnp.int32)
                band = bits & mask7f
                bins = lax.shift_right_logical(band, 16)
                plsc.addupdate_scatter(hist, [bins], ones)

        for s in range(RING):
            pltpu.async_copy(w_hbm.at[row0 + s], bufs[s], sems[s])

        def outer(i, _):
            for s in range(RING):
                row = i * RING + s
                pltpu.make_async_copy(w_hbm.at[row0], bufs[s], sems[s]).wait()
                process(bufs[s])

                @pl.when(row + RING < ROWS_PER_TILE)
                def _():
                    pltpu.async_copy(
                        w_hbm.at[row0 + row + RING], bufs[s], sems[s])
            return 0

        lax.fori_loop(0, ROWS_PER_TILE // RING, outer, 0)
        pltpu.sync_copy(hist, out_hbm.at[wid])

    return k(w2d)


def _rank_search_top(hist3):
    def body(h_ref, b0_ref, rr0_ref):
        h = jnp.sum(h_ref[...], axis=0)
        c = h
        s = 1
        while s < 128:
            c = c + jnp.concatenate(
                [jnp.zeros((256, s), jnp.int32), c[:, :-s]], axis=1)
            s *= 2
        rt = c[:, 127:128]
        e = rt
        s = 1
        while s < 256:
            e = e + jnp.concatenate(
                [jnp.zeros((s, 1), jnp.int32), e[:-s, :]], axis=0)
            s *= 2
        cum = c + (e - rt)
        b0 = jnp.sum((cum <= RANK).astype(jnp.int32))
        row = lax.broadcasted_iota(jnp.int32, (256, 128), 0)
        col = lax.broadcasted_iota(jnp.int32, (256, 128), 1)
        binidx = row * 128 + col
        cumexcl = cum - h
        rr0 = RANK - jnp.sum(jnp.where(binidx == b0, cumexcl, 0))
        b0_ref[...] = jnp.full((1, 128), b0, jnp.int32)
        rr0_ref[...] = jnp.full((1, 128), rr0, jnp.int32)

    return pl.pallas_call(
        body,
        out_shape=[jax.ShapeDtypeStruct((1, 128), jnp.int32),
                   jax.ShapeDtypeStruct((1, 128), jnp.int32)],
    )(hist3)


def _hist_low(w2d, b0_16):
    @functools.partial(
        pl.kernel,
        mesh=_sc_mesh(),
        out_type=jax.ShapeDtypeStruct((NW, LOW_BINS), jnp.int32),
        scratch_types=(
            [pltpu.VMEM((4096,), jnp.float32)] * RING2
            + [pltpu.VMEM((LOW_BINS,), jnp.int32),
               pltpu.VMEM((16,), jnp.int32)]
            + [pltpu.SemaphoreType.DMA] * RING2
        ),
        compiler_params=_SC_PARAMS,
    )
    def k(w_hbm, b0_hbm, out_hbm, *scratch):
        bufs = scratch[:RING2]
        hist = scratch[RING2]
        b0v = scratch[RING2 + 1]
        sems = scratch[RING2 + 2:]
        wid = lax.axis_index("s") * 2 + lax.axis_index("c")
        row0 = wid * ROWS_PER_TILE
        pltpu.sync_copy(b0_hbm, b0v)
        bv = b0v[...]

        @plsc.parallel_loop(0, LOW_BINS // 16, unroll=8)
        def _zero(i):
            hist[pl.ds(i * 16, 16)] = jnp.zeros((16,), jnp.int32)

        ones = jnp.ones((16,), jnp.int32)
        mask7f = jnp.full((16,), 0x7FFFFFFF, jnp.int32)
        maskff = jnp.full((16,), 0xFFFF, jnp.int32)

        def process(buf):
            @plsc.parallel_loop(0, 4096 // 16, unroll=16)
            def _body(j):
                bits = plsc.bitcast(buf[pl.ds(j * 16, 16)], jnp.int32)
                band = bits & mask7f
                top = lax.shift_right_logical(band, 16)
                low = band & maskff
                m = top == bv
                plsc.addupdate_scatter(hist, [low], ones, mask=m)

        for s in range(RING2):
            pltpu.async_copy(w_hbm.at[row0 + s], bufs[s], sems[s])

        def outer(i, _):
            for s in range(RING2):
                row = i * RING2 + s
                pltpu.make_async_copy(w_hbm.at[row0], bufs[s], sems[s]).wait()
                process(bufs[s])

                @pl.when(row + RING2 < ROWS_PER_TILE)
                def _():
                    pltpu.async_copy(
                        w_hbm.at[row0 + row + RING2], bufs[s], sems[s])
            return 0

        lax.fori_loop(0, ROWS_PER_TILE // RING2, outer, 0)
        pltpu.sync_copy(hist, out_hbm.at[wid])

    return k(w2d, b0_16)


def _rank_search_low(hist3, b0_row, rr0_row):
    def body(h_ref, b0_ref, rr0_ref, thr_ref):
        h = jnp.sum(h_ref[...], axis=0)
        c = h
        s = 1
        while s < 128:
            c = c + jnp.concatenate(
                [jnp.zeros((512, s), jnp.int32), c[:, :-s]], axis=1)
            s *= 2
        rt = c[:, 127:128]
        e = rt
        s = 1
        while s < 512:
            e = e + jnp.concatenate(
                [jnp.zeros((s, 1), jnp.int32), e[:-s, :]], axis=0)
            s *= 2
        cum = c + (e - rt)
        rr0 = jnp.max(rr0_ref[...])
        b0 = jnp.max(b0_ref[...])
        low0 = jnp.sum((cum <= rr0).astype(jnp.int32))
        pattern = lax.shift_left(b0, 16) | low0
        thr_ref[...] = lax.bitcast_convert_type(
            jnp.full((1, 128), pattern, jnp.int32), jnp.float32)

    return pl.pallas_call(
        body,
        out_shape=jax.ShapeDtypeStruct((1, 128), jnp.float32),
    )(hist3, b0_row, rr0_row)


def _masked_matmul(x, w, bias2d, thr_row):
    BLK = 256
    grid = 4096 // BLK

    def body(thr_ref, x_ref, w_ref, b_ref, o_ref):
        t = jnp.max(thr_ref[...])
        w_blk = w_ref[...]
        wm = jnp.where(jnp.abs(w_blk) >= t, w_blk, 0.0)
        y = lax.dot_general(
            x_ref[...], wm, (((1,), (1,)), ((), ())),
            preferred_element_type=jnp.float32,
            precision=lax.Precision.DEFAULT)
        o_ref[...] = y + b_ref[...]

    return pl.pallas_call(
        body,
        grid=(grid,),
        in_specs=[
            pl.BlockSpec((1, 128), lambda i: (0, 0)),
            pl.BlockSpec((32, 4096), lambda i: (0, 0)),
            pl.BlockSpec((BLK, 4096), lambda i: (i, 0)),
            pl.BlockSpec((1, BLK), lambda i: (0, i)),
        ],
        out_specs=pl.BlockSpec((32, BLK), lambda i: (0, i)),
        out_shape=jax.ShapeDtypeStruct((32, 4096), jnp.float32),
    )(thr_row, x, w, bias2d)


def kernel(inputs, weight, bias):
    hist1 = _hist_top(weight)
    b0_row, rr0_row = _rank_search_top(hist1.reshape(NW, TOP_BINS // 128, 128))
    b0_16 = b0_row[0, :16]
    hist2 = _hist_low(weight, b0_16)
    thr_row = _rank_search_low(
        hist2.reshape(NW, LOW_BINS // 128, 128), b0_row, rr0_row)
    return _masked_matmul(inputs, weight, bias.reshape(1, 4096), thr_row)


# speedup vs baseline: 1.1147x; 1.1147x over previous
import functools

import jax
import jax.numpy as jnp
from jax import lax
from jax.experimental import pallas as pl
from jax.experimental.pallas import tpu as pltpu
from jax.experimental.pallas import tpu_sc as plsc

N_ELEMS = 4096 * 4096
RANK = 15099493
NW = 32
SHARD = N_ELEMS // NW

TOP_BINS = 32768
LOW_BINS = 65536

_SC_PARAMS = pltpu.CompilerParams(needs_layout_passes=False)


def _sc_mesh():
    return plsc.VectorSubcoreMesh(core_axis_name="c", subcore_axis_name="s")


ROWS_PER_TILE = SHARD // 4096
RING = 8
RING2 = 8


def _hist_top(w2d):
    @functools.partial(
        pl.kernel,
        mesh=_sc_mesh(),
        out_type=jax.ShapeDtypeStruct((NW, TOP_BINS), jnp.int32),
        scratch_types=(
            [pltpu.VMEM((4096,), jnp.float32)] * RING
            + [pltpu.VMEM((TOP_BINS,), jnp.int32)]
            + [pltpu.SemaphoreType.DMA] * RING
        ),
        compiler_params=_SC_PARAMS,
    )
    def k(w_hbm, out_hbm, *scratch):
        bufs = scratch[:RING]
        hist = scratch[RING]
        sems = scratch[RING + 1:]
        wid = lax.axis_index("s") * 2 + lax.axis_index("c")
        row0 = wid * ROWS_PER_TILE

        @plsc.parallel_loop(0, TOP_BINS // 16, unroll=8)
        def _zero(i):
            hist[pl.ds(i * 16, 16)] = jnp.zeros((16,), jnp.int32)

        ones = jnp.ones((16,), jnp.int32)
        mask7f = jnp.full((16,), 0x7FFFFFFF, jnp.int32)

        def process(buf):
            @plsc.parallel_loop(0, 4096 // 16, unroll=8)
            def _body(j):
                bits = plsc.bitcast(buf[pl.ds(j * 16, 16)], jnp.int32)
                band = bits & mask7f
                bins = lax.shift_right_logical(band, 16)
                plsc.addupdate_scatter(hist, [bins], ones)

        for s in range(RING):
            pltpu.async_copy(w_hbm.at[row0 + s], bufs[s], sems[s])

        def outer(i, _):
            for s in range(RING):
                row = i * RING + s
                pltpu.make_async_copy(w_hbm.at[row0], bufs[s], sems[s]).wait()
                process(bufs[s])

                @pl.when(row + RING < ROWS_PER_TILE)
                def _():
                    pltpu.async_copy(
                        w_hbm.at[row0 + row + RING], bufs[s], sems[s])
            return 0

        lax.fori_loop(0, ROWS_PER_TILE // RING, outer, 0)
        pltpu.sync_copy(hist, out_hbm.at[wid])

    return k(w2d)


def _rank_search_top(hist3):
    def body(h_ref, b0_ref, rr0_ref):
        h = jnp.sum(h_ref[...], axis=0)
        c = h
        s = 1
        while s < 128:
            c = c + jnp.concatenate(
                [jnp.zeros((256, s), jnp.int32), c[:, :-s]], axis=1)
            s *= 2
        rt = c[:, 127:128]
        e = rt
        s = 1
        while s < 256:
            e = e + jnp.concatenate(
                [jnp.zeros((s, 1), jnp.int32), e[:-s, :]], axis=0)
            s *= 2
        cum = c + (e - rt)
        b0 = jnp.sum((cum <= RANK).astype(jnp.int32))
        row = lax.broadcasted_iota(jnp.int32, (256, 128), 0)
        col = lax.broadcasted_iota(jnp.int32, (256, 128), 1)
        binidx = row * 128 + col
        cumexcl = cum - h
        rr0 = RANK - jnp.sum(jnp.where(binidx == b0, cumexcl, 0))
        b0_ref[...] = jnp.full((1, 128), b0, jnp.int32)
        rr0_ref[...] = jnp.full((1, 128), rr0, jnp.int32)

    return pl.pallas_call(
        body,
        out_shape=[jax.ShapeDtypeStruct((1, 128), jnp.int32),
                   jax.ShapeDtypeStruct((1, 128), jnp.int32)],
    )(hist3)


def _hist_low(w2d, b0_16):
    @functools.partial(
        pl.kernel,
        mesh=_sc_mesh(),
        out_type=jax.ShapeDtypeStruct((NW, LOW_BINS), jnp.int32),
        scratch_types=(
            [pltpu.VMEM((4096,), jnp.float32)] * RING2
            + [pltpu.VMEM((LOW_BINS,), jnp.int32),
               pltpu.VMEM((16,), jnp.int32)]
            + [pltpu.SemaphoreType.DMA] * RING2
        ),
        compiler_params=_SC_PARAMS,
    )
    def k(w_hbm, b0_hbm, out_hbm, *scratch):
        bufs = scratch[:RING2]
        hist = scratch[RING2]
        b0v = scratch[RING2 + 1]
        sems = scratch[RING2 + 2:]
        wid = lax.axis_index("s") * 2 + lax.axis_index("c")
        row0 = wid * ROWS_PER_TILE
        pltpu.sync_copy(b0_hbm, b0v)
        bv = b0v[...]

        @plsc.parallel_loop(0, LOW_BINS // 16, unroll=8)
        def _zero(i):
            hist[pl.ds(i * 16, 16)] = jnp.zeros((16,), jnp.int32)

        ones = jnp.ones((16,), jnp.int32)
        mask7f = jnp.full((16,), 0x7FFFFFFF, jnp.int32)
        maskff = jnp.full((16,), 0xFFFF, jnp.int32)

        def process(buf):
            @plsc.parallel_loop(0, 4096 // 16, unroll=8)
            def _body(j):
                bits = plsc.bitcast(buf[pl.ds(j * 16, 16)], jnp.int32)
                band = bits & mask7f
                top = lax.shift_right_logical(band, 16)
                low = band & maskff
                m = top == bv
                plsc.addupdate_scatter(hist, [low], ones, mask=m)

        for s in range(RING2):
            pltpu.async_copy(w_hbm.at[row0 + s], bufs[s], sems[s])

        def outer(i, _):
            for s in range(RING2):
                row = i * RING2 + s
                pltpu.make_async_copy(w_hbm.at[row0], bufs[s], sems[s]).wait()
                process(bufs[s])

                @pl.when(row + RING2 < ROWS_PER_TILE)
                def _():
                    pltpu.async_copy(
                        w_hbm.at[row0 + row + RING2], bufs[s], sems[s])
            return 0

        lax.fori_loop(0, ROWS_PER_TILE // RING2, outer, 0)
        pltpu.sync_copy(hist, out_hbm.at[wid])

    return k(w2d, b0_16)


def _rank_search_low(hist3, b0_row, rr0_row):
    def body(h_ref, b0_ref, rr0_ref, thr_ref):
        h = jnp.sum(h_ref[...], axis=0)
        c = h
        s = 1
        while s < 128:
            c = c + jnp.concatenate(
                [jnp.zeros((512, s), jnp.int32), c[:, :-s]], axis=1)
            s *= 2
        rt = c[:, 127:128]
        e = rt
        s = 1
        while s < 512:
            e = e + jnp.concatenate(
                [jnp.zeros((s, 1), jnp.int32), e[:-s, :]], axis=0)
            s *= 2
        cum = c + (e - rt)
        rr0 = jnp.max(rr0_ref[...])
        b0 = jnp.max(b0_ref[...])
        low0 = jnp.sum((cum <= rr0).astype(jnp.int32))
        pattern = lax.shift_left(b0, 16) | low0
        thr_ref[...] = lax.bitcast_convert_type(
            jnp.full((1, 128), pattern, jnp.int32), jnp.float32)

    return pl.pallas_call(
        body,
        out_shape=jax.ShapeDtypeStruct((1, 128), jnp.float32),
    )(hist3, b0_row, rr0_row)


def _masked_matmul(x, w, bias2d, thr_row):
    BLK = 256
    grid = 4096 // BLK

    def body(thr_ref, x_ref, w_ref, b_ref, o_ref):
        t = jnp.max(thr_ref[...])
        w_blk = w_ref[...]
        wm = jnp.where(jnp.abs(w_blk) >= t, w_blk, 0.0)
        y = lax.dot_general(
            x_ref[...], wm, (((1,), (1,)), ((), ())),
            preferred_element_type=jnp.float32,
            precision=lax.Precision.DEFAULT)
        o_ref[...] = y + b_ref[...]

    return pl.pallas_call(
        body,
        grid=(grid,),
        in_specs=[
            pl.BlockSpec((1, 128), lambda i: (0, 0)),
            pl.BlockSpec((32, 4096), lambda i: (0, 0)),
            pl.BlockSpec((BLK, 4096), lambda i: (i, 0)),
            pl.BlockSpec((1, BLK), lambda i: (0, i)),
        ],
        out_specs=pl.BlockSpec((32, BLK), lambda i: (0, i)),
        out_shape=jax.ShapeDtypeStruct((32, 4096), jnp.float32),
    )(thr_row, x, w, bias2d)


def kernel(inputs, weight, bias):
    hist1 = _hist_top(weight)
    b0_row, rr0_row = _rank_search_top(hist1.reshape(NW, TOP_BINS // 128, 128))
    b0_16 = b0_row[0, :16]
    hist2 = _hist_low(weight, b0_16)
    thr_row = _rank_search_low(
        hist2.reshape(NW, LOW_BINS // 128, 128), b0_row, rr0_row)
    return _masked_matmul(inputs, weight, bias.reshape(1, 4096), thr_row)


# speedup vs baseline: 1.1201x; 1.0049x over previous
import functools

import jax
import jax.numpy as jnp
from jax import lax
from jax.experimental import pallas as pl
from jax.experimental.pallas import tpu as pltpu
from jax.experimental.pallas import tpu_sc as plsc

N_ELEMS = 4096 * 4096
RANK = 15099493
NW = 32
SHARD = N_ELEMS // NW

TOP_BINS = 32768
LOW_BINS = 65536

_SC_PARAMS = pltpu.CompilerParams(needs_layout_passes=False)


def _sc_mesh():
    return plsc.VectorSubcoreMesh(core_axis_name="c", subcore_axis_name="s")


ROWS_PER_TILE = SHARD // 4096
RING = 8
RING2 = 8


def _hist_top(w2d):
    @functools.partial(
        pl.kernel,
        mesh=_sc_mesh(),
        out_type=jax.ShapeDtypeStruct((NW, TOP_BINS), jnp.int32),
        scratch_types=(
            [pltpu.VMEM((4096,), jnp.float32)] * RING
            + [pltpu.VMEM((TOP_BINS,), jnp.int32)]
            + [pltpu.SemaphoreType.DMA] * RING
        ),
        compiler_params=_SC_PARAMS,
    )
    def k(w_hbm, out_hbm, *scratch):
        bufs = scratch[:RING]
        hist = scratch[RING]
        sems = scratch[RING + 1:]
        wid = lax.axis_index("s") * 2 + lax.axis_index("c")
        row0 = wid * ROWS_PER_TILE

        @plsc.parallel_loop(0, TOP_BINS // 16, unroll=8)
        def _zero(i):
            hist[pl.ds(i * 16, 16)] = jnp.zeros((16,), jnp.int32)

        ones = jnp.ones((16,), jnp.int32)
        mask7f = jnp.full((16,), 0x7FFFFFFF, jnp.int32)

        def process(buf):
            @plsc.parallel_loop(0, 4096 // 16, unroll=8)
            def _body(j):
                bits = plsc.bitcast(buf[pl.ds(j * 16, 16)], jnp.int32)
                band = bits & mask7f
                bins = lax.shift_right_logical(band, 16)
                plsc.addupdate_scatter(hist, [bins], ones)

        for s in range(RING):
            pltpu.async_copy(w_hbm.at[row0 + s], bufs[s], sems[s])

        def outer(i, _):
            for s in range(RING):
                row = i * RING + s
                pltpu.make_async_copy(w_hbm.at[row0], bufs[s], sems[s]).wait()
                process(bufs[s])

                @pl.when(row + RING < ROWS_PER_TILE)
                def _():
                    pltpu.async_copy(
                        w_hbm.at[row0 + row + RING], bufs[s], sems[s])
            return 0

        lax.fori_loop(0, ROWS_PER_TILE // RING, outer, 0)
        pltpu.sync_copy(hist, out_hbm.at[wid])

    return k(w2d)


def _rank_search_top(hist3):
    def body(h_ref, b0_ref, rr0_ref):
        h = jnp.sum(h_ref[...], axis=0)
        c = h
        s = 1
        while s < 128:
            c = c + jnp.concatenate(
                [jnp.zeros((256, s), jnp.int32), c[:, :-s]], axis=1)
            s *= 2
        rt = c[:, 127:128]
        e = rt
        s = 1
        while s < 256:
            e = e + jnp.concatenate(
                [jnp.zeros((s, 1), jnp.int32), e[:-s, :]], axis=0)
            s *= 2
        cum = c + (e - rt)
        b0 = jnp.sum((cum <= RANK).astype(jnp.int32))
        row = lax.broadcasted_iota(jnp.int32, (256, 128), 0)
        col = lax.broadcasted_iota(jnp.int32, (256, 128), 1)
        binidx = row * 128 + col
        cumexcl = cum - h
        rr0 = RANK - jnp.sum(jnp.where(binidx == b0, cumexcl, 0))
        b0_ref[...] = jnp.full((1, 128), b0, jnp.int32)
        rr0_ref[...] = jnp.full((1, 128), rr0, jnp.int32)

    return pl.pallas_call(
        body,
        out_shape=[jax.ShapeDtypeStruct((1, 128), jnp.int32),
                   jax.ShapeDtypeStruct((1, 128), jnp.int32)],
    )(hist3)


def _hist_low(w2d, b0_16):
    @functools.partial(
        pl.kernel,
        mesh=_sc_mesh(),
        out_type=jax.ShapeDtypeStruct((NW, LOW_BINS), jnp.int32),
        scratch_types=(
            [pltpu.VMEM((4096,), jnp.float32)] * RING2
            + [pltpu.VMEM((LOW_BINS,), jnp.int32),
               pltpu.VMEM((16,), jnp.int32)]
            + [pltpu.SemaphoreType.DMA] * RING2
        ),
        compiler_params=_SC_PARAMS,
    )
    def k(w_hbm, b0_hbm, out_hbm, *scratch):
        bufs = scratch[:RING2]
        hist = scratch[RING2]
        b0v = scratch[RING2 + 1]
        sems = scratch[RING2 + 2:]
        wid = lax.axis_index("s") * 2 + lax.axis_index("c")
        row0 = wid * ROWS_PER_TILE
        pltpu.sync_copy(b0_hbm, b0v)
        bv = b0v[...]

        @plsc.parallel_loop(0, LOW_BINS // 16, unroll=8)
        def _zero(i):
            hist[pl.ds(i * 16, 16)] = jnp.zeros((16,), jnp.int32)

        ones = jnp.ones((16,), jnp.int32)
        mask7f = jnp.full((16,), 0x7FFFFFFF, jnp.int32)
        maskff = jnp.full((16,), 0xFFFF, jnp.int32)

        def process(buf):
            @plsc.parallel_loop(0, 4096 // 16, unroll=8)
            def _body(j):
                bits = plsc.bitcast(buf[pl.ds(j * 16, 16)], jnp.int32)
                band = bits & mask7f
                top = lax.shift_right_logical(band, 16)
                low = band & maskff
                m = top == bv
                plsc.addupdate_scatter(hist, [low], ones, mask=m)

        for s in range(RING2):
            pltpu.async_copy(w_hbm.at[row0 + s], bufs[s], sems[s])

        def outer(i, _):
            for s in range(RING2):
                row = i * RING2 + s
                pltpu.make_async_copy(w_hbm.at[row0], bufs[s], sems[s]).wait()
                process(bufs[s])

                @pl.when(row + RING2 < ROWS_PER_TILE)
                def _():
                    pltpu.async_copy(
                        w_hbm.at[row0 + row + RING2], bufs[s], sems[s])
            return 0

        lax.fori_loop(0, ROWS_PER_TILE // RING2, outer, 0)
        pltpu.sync_copy(hist, out_hbm.at[wid])

    return k(w2d, b0_16)


def _rank_search_low(hist3, b0_row, rr0_row):
    def body(h_ref, b0_ref, rr0_ref, thr_ref):
        h = jnp.sum(h_ref[...], axis=0)
        c = h
        s = 1
        while s < 128:
            c = c + jnp.concatenate(
                [jnp.zeros((512, s), jnp.int32), c[:, :-s]], axis=1)
            s *= 2
        rt = c[:, 127:128]
        e = rt
        s = 1
        while s < 512:
            e = e + jnp.concatenate(
                [jnp.zeros((s, 1), jnp.int32), e[:-s, :]], axis=0)
            s *= 2
        cum = c + (e - rt)
        rr0 = jnp.max(rr0_ref[...])
        b0 = jnp.max(b0_ref[...])
        low0 = jnp.sum((cum <= rr0).astype(jnp.int32))
        pattern = lax.shift_left(b0, 16) | low0
        thr_ref[...] = lax.bitcast_convert_type(
            jnp.full((1, 128), pattern, jnp.int32), jnp.float32)

    return pl.pallas_call(
        body,
        out_shape=jax.ShapeDtypeStruct((1, 128), jnp.float32),
    )(hist3, b0_row, rr0_row)


def _masked_matmul(x, w, bias2d, thr_row):
    BLK = 1024
    grid = 4096 // BLK

    def body(thr_ref, x_ref, w_ref, b_ref, o_ref):
        t = jnp.max(thr_ref[...])
        w_blk = w_ref[...]
        wm = jnp.where(jnp.abs(w_blk) >= t, w_blk, 0.0)
        y = lax.dot_general(
            x_ref[...], wm, (((1,), (1,)), ((), ())),
            preferred_element_type=jnp.float32,
            precision=lax.Precision.DEFAULT)
        o_ref[...] = y + b_ref[...]

    return pl.pallas_call(
        body,
        grid=(grid,),
        in_specs=[
            pl.BlockSpec((1, 128), lambda i: (0, 0)),
            pl.BlockSpec((32, 4096), lambda i: (0, 0)),
            pl.BlockSpec((BLK, 4096), lambda i: (i, 0)),
            pl.BlockSpec((1, BLK), lambda i: (0, i)),
        ],
        out_specs=pl.BlockSpec((32, BLK), lambda i: (0, i)),
        out_shape=jax.ShapeDtypeStruct((32, 4096), jnp.float32),
    )(thr_row, x, w, bias2d)


def kernel(inputs, weight, bias):
    hist1 = _hist_top(weight)
    b0_row, rr0_row = _rank_search_top(hist1.reshape(NW, TOP_BINS // 128, 128))
    b0_16 = b0_row[0, :16]
    hist2 = _hist_low(weight, b0_16)
    thr_row = _rank_search_low(
        hist2.reshape(NW, LOW_BINS // 128, 128), b0_row, rr0_row)
    return _masked_matmul(inputs, weight, bias.reshape(1, 4096), thr_row)
